# baseline (speedup 1.0000x reference)
import dataclasses
import functools

import jax
import jax.numpy as jnp
from jax import lax
from jax.experimental import pallas as pl
from jax.experimental.pallas import tpu as pltpu
from jax.experimental.pallas import tpu_sc as plsc

NC = 2
NS = 16
L = 16
CHUNK = 128
DEG_W = 16
G = 4


def _sc_aggregate(xall, src_p, dst_p, zeros_feat, n_pad, cps, dh):
    mesh = plsc.VectorSubcoreMesh(core_axis_name="c", subcore_axis_name="s")
    rpw = n_pad // NS

    full_blks, tail = rpw // CHUNK, rpw % CHUNK
    blks = [(k * CHUNK, CHUNK) for k in range(full_blks)]
    if tail:
        blks.append((full_blks * CHUNK, tail))

    cp = pltpu.CompilerParams()
    if "needs_layout_passes" in pltpu.CompilerParams.__dataclass_fields__:
        cp = dataclasses.replace(cp, needs_layout_passes=False)

    @functools.partial(
        pl.kernel,
        compiler_params=cp,
        out_type=[
            jax.ShapeDtypeStruct((NC * n_pad, dh), jnp.float32),
            jax.ShapeDtypeStruct((NC * NS, n_pad), jnp.float32),
        ],
        mesh=mesh,
        scratch_types=[
            pltpu.VMEM((G, CHUNK), jnp.int32),
            pltpu.VMEM((2, G, CHUNK), jnp.int32),
            pltpu.VMEM((2, CHUNK, dh), jnp.float32),
            pltpu.VMEM((n_pad,), jnp.float32),
            pltpu.VMEM_SHARED((n_pad, dh), jnp.float32),
            pltpu.SemaphoreType.DMA,
            pltpu.SemaphoreType.DMA,
            pltpu.SemaphoreType.DMA,
            pltpu.SemaphoreType.DMA,
        ],
    )
    def k(xall_hbm, src_hbm, dst_hbm, zf_hbm,
          sum_out, parts_out,
          sidx_v, didx_v, rows_v, dloc_v,
          acc, gsem0, gsem1, ssem0, ssem1):
        c = lax.axis_index("c")
        s = lax.axis_index("s")
        base = s * rpw
        obase = c * n_pad + s * rpw
        ones16 = jnp.full((L,), 1.0, jnp.float32)

        zero16 = jnp.zeros((L,), jnp.float32)

        @pl.loop(0, n_pad, step=L)
        def _(i):
            dloc_v[pl.ds(i, L)] = zero16

        pltpu.sync_copy(zf_hbm, rows_v.at[0])
        for off, nrows in blks:
            pltpu.sync_copy(rows_v.at[0].at[pl.ds(0, nrows)],
                            acc.at[pl.ds(base + off, nrows)])
        plsc.subcore_barrier()

        gsems = (gsem0, gsem1)
        ssems = (ssem0, ssem1)
        ng = cps // G

        def wait_gather(j, b):
            pltpu.make_async_copy(xall_hbm.at[sidx_v.at[j]],
                                  rows_v.at[b], gsems[b]).wait()

        def wait_scatter(p, b):
            pltpu.make_async_copy(rows_v.at[b],
                                  acc.at[didx_v.at[p].at[0]],
                                  ssems[b]).wait()

        pltpu.sync_copy(src_hbm.at[c].at[s].at[pl.ds(0, G)], sidx_v)
        pltpu.sync_copy(dst_hbm.at[s].at[pl.ds(0, G)], didx_v.at[0])
        pltpu.async_copy(xall_hbm.at[sidx_v.at[0]], rows_v.at[0], gsems[0])

        @pl.loop(0, ng)
        def _(g):
            p = g % 2
            for j in range(G):
                b = j % 2
                if j + 1 < G:
                    if j == 0:
                        @pl.when(g > 0)
                        def _():
                            wait_scatter(1 - p, 1 - b)
                    else:
                        wait_scatter(p, 1 - b)
                    pltpu.async_copy(xall_hbm.at[sidx_v.at[j + 1]],
                                     rows_v.at[1 - b], gsems[1 - b])
                wait_gather(j, b)
                pltpu.async_copy(rows_v.at[b], acc.at[didx_v.at[p].at[j]],
                                 ssems[b], add=True)
                if j + 1 == G:
                    @pl.when(g + 1 < ng)
                    def _():
                        pltpu.sync_copy(
                            src_hbm.at[c].at[s].at[pl.ds((g + 1) * G, G)],
                            sidx_v)
                        pltpu.sync_copy(
                            dst_hbm.at[s].at[pl.ds((g + 1) * G, G)],
                            didx_v.at[1 - p])
                        wait_scatter(p, 1 - b)
                        pltpu.async_copy(xall_hbm.at[sidx_v.at[0]],
                                         rows_v.at[1 - b], gsems[1 - b])
                for q in range(CHUNK // L):
                    idx16 = didx_v[p, j, pl.ds(q * L, L)]
                    plsc.addupdate_scatter(dloc_v, [idx16], ones16)

        wait_scatter((ng - 1) % 2, (G - 2) % 2)
        wait_scatter((ng - 1) % 2, (G - 1) % 2)

        pltpu.sync_copy(dloc_v, parts_out.at[c * NS + s])
        plsc.subcore_barrier()

        for off, nrows in blks:
            pltpu.sync_copy(acc.at[pl.ds(base + off, nrows)],
                            rows_v.at[0].at[pl.ds(0, nrows)])
            pltpu.sync_copy(rows_v.at[0].at[pl.ds(0, nrows)],
                            sum_out.at[pl.ds(obase + off, nrows)])

    return k(xall, src_p, dst_p, zeros_feat)


def _tc_root(x, wrt, b2, n, d_out, d_in):
    bm = 1024
    grid = ((n + bm - 1) // bm,)

    def body(x_ref, wrt_ref, b_ref, o_ref):
        o_ref[...] = (jnp.dot(x_ref[...], wrt_ref[...],
                              preferred_element_type=jnp.float32)
                      + b_ref[...])

    return pl.pallas_call(
        body,
        grid=grid,
        in_specs=[
            pl.BlockSpec((bm, d_in), lambda i: (i, 0)),
            pl.BlockSpec((d_in, d_out), lambda i: (0, 0)),
            pl.BlockSpec((1, d_out), lambda i: (0, 0)),
        ],
        out_specs=pl.BlockSpec((bm, d_out), lambda i: (i, 0)),
        out_shape=jax.ShapeDtypeStruct((n, d_out), jnp.float32),
    )(x, wrt, b2)


def _tc_combine(xr, summed, degs, wl0, wl1, n, d_out, dh):
    bm = 1024
    grid = ((n + bm - 1) // bm,)

    def body(xr_ref, s0_ref, s1_ref, p_ref,
             wl0_ref, wl1_ref, o_ref):
        deg = lax.dot_general(p_ref[...], jnp.ones((NS, 1), jnp.float32),
                              dimension_numbers=(((0,), (0,)), ((), ())),
                              preferred_element_type=jnp.float32)
        r = 1.0 / jnp.maximum(deg, 1.0)
        a0 = s0_ref[0] * r
        a1 = s1_ref[0] * r
        z = (jnp.dot(a0, wl0_ref[...], preferred_element_type=jnp.float32)
             + jnp.dot(a1, wl1_ref[...], preferred_element_type=jnp.float32)
             + xr_ref[...])
        o_ref[...] = jnp.maximum(z, 0.0)

    return pl.pallas_call(
        body,
        grid=grid,
        in_specs=[
            pl.BlockSpec((bm, d_out), lambda i: (i, 0)),
            pl.BlockSpec((1, bm, dh), lambda i: (0, i, 0)),
            pl.BlockSpec((1, bm, dh), lambda i: (1, i, 0)),
            pl.BlockSpec((NS, bm), lambda i: (0, i)),
            pl.BlockSpec((dh, d_out), lambda i: (0, 0)),
            pl.BlockSpec((dh, d_out), lambda i: (0, 0)),
        ],
        out_specs=pl.BlockSpec((bm, d_out), lambda i: (i, 0)),
        out_shape=jax.ShapeDtypeStruct((n, d_out), jnp.float32),
    )(xr, summed, summed, degs, wl0, wl1)


def kernel(x, edge_index, W_l, b_l, W_r):
    n, d_in = x.shape
    d_out = W_l.shape[0]
    dh = d_in // 2
    e = edge_index.shape[1]

    eps = ((e + NS * G * CHUNK - 1) // (NS * G * CHUNK)) * G * CHUNK
    e_pad = eps * NS
    cps = eps // CHUNK
    n_pad = ((n + 1 + NS * CHUNK - 1) // (NS * CHUNK)) * (NS * CHUNK)

    src = edge_index[0].astype(jnp.int32)
    dst = edge_index[1].astype(jnp.int32)
    pad = e_pad - e
    src_flat = jnp.concatenate([src, jnp.zeros((pad,), jnp.int32)])
    dst_flat = jnp.concatenate([dst, jnp.full((pad,), n, jnp.int32)])
    src_p = jnp.stack([2 * src_flat, 2 * src_flat + 1]).reshape(
        NC, NS, cps, CHUNK)
    dst_p = dst_flat.reshape(NS, cps, CHUNK)
    xview = x.reshape(NC * n, dh)

    zeros_feat = jnp.zeros((CHUNK, dh), jnp.float32)

    summed, parts = _sc_aggregate(xview, src_p, dst_p, zeros_feat,
                                  n_pad, cps, dh)
    summed = summed.reshape(NC, n_pad, dh)

    wl0 = W_l[:, :dh].T
    wl1 = W_l[:, dh:].T
    wrt = W_r.T
    b2 = b_l.reshape(1, d_out)
    xr = _tc_root(x, wrt, b2, n, d_out, d_in)
    return _tc_combine(xr, summed, parts, wl0, wl1, n, d_out, dh)


# speedup vs baseline: 1.0061x; 1.0061x over previous
import dataclasses
import functools

import jax
import jax.numpy as jnp
from jax import lax
from jax.experimental import pallas as pl
from jax.experimental.pallas import tpu as pltpu
from jax.experimental.pallas import tpu_sc as plsc

NC = 2
NS = 16
L = 16
CHUNK = 128
DEG_W = 16
G = 4


def _sc_aggregate(xall, src_p, dst_p, zeros_feat, n_pad, cps, dh):
    mesh = plsc.VectorSubcoreMesh(core_axis_name="c", subcore_axis_name="s")
    rpw = n_pad // NS

    full_blks, tail = rpw // CHUNK, rpw % CHUNK
    blks = [(k * CHUNK, CHUNK) for k in range(full_blks)]
    if tail:
        blks.append((full_blks * CHUNK, tail))

    cp = pltpu.CompilerParams()
    if "needs_layout_passes" in pltpu.CompilerParams.__dataclass_fields__:
        cp = dataclasses.replace(cp, needs_layout_passes=False)

    @functools.partial(
        pl.kernel,
        compiler_params=cp,
        out_type=[
            jax.ShapeDtypeStruct((NC * n_pad, dh), jnp.float32),
            jax.ShapeDtypeStruct((NC * NS, n_pad), jnp.float32),
        ],
        mesh=mesh,
        scratch_types=[
            pltpu.VMEM((G, CHUNK), jnp.int32),
            pltpu.VMEM((2, G, CHUNK), jnp.int32),
            pltpu.VMEM((2, CHUNK, dh), jnp.float32),
            pltpu.VMEM((n_pad,), jnp.float32),
            pltpu.VMEM_SHARED((n_pad, dh), jnp.float32),
            pltpu.SemaphoreType.DMA,
            pltpu.SemaphoreType.DMA,
            pltpu.SemaphoreType.DMA,
            pltpu.SemaphoreType.DMA,
        ],
    )
    def k(xall_hbm, src_hbm, dst_hbm, zf_hbm,
          sum_out, parts_out,
          sidx_v, didx_v, rows_v, dloc_v,
          acc, gsem0, gsem1, ssem0, ssem1):
        c = lax.axis_index("c")
        s = lax.axis_index("s")
        base = s * rpw
        obase = c * n_pad + s * rpw
        ones16 = jnp.full((L,), 1.0, jnp.float32)

        zero16 = jnp.zeros((L,), jnp.float32)

        @pl.loop(0, n_pad, step=L)
        def _(i):
            dloc_v[pl.ds(i, L)] = zero16

        pltpu.sync_copy(zf_hbm, rows_v.at[0])
        for off, nrows in blks:
            pltpu.sync_copy(rows_v.at[0].at[pl.ds(0, nrows)],
                            acc.at[pl.ds(base + off, nrows)])
        plsc.subcore_barrier()

        gsems = (gsem0, gsem1)
        ssems = (ssem0, ssem1)
        ng = cps // G

        def wait_gather(j, b):
            pltpu.make_async_copy(xall_hbm.at[sidx_v.at[j]],
                                  rows_v.at[b], gsems[b]).wait()

        def wait_scatter(p, b):
            pltpu.make_async_copy(rows_v.at[b],
                                  acc.at[didx_v.at[p].at[0]],
                                  ssems[b]).wait()

        pltpu.sync_copy(src_hbm.at[c].at[s].at[pl.ds(0, G)], sidx_v)
        pltpu.sync_copy(dst_hbm.at[s].at[pl.ds(0, G)], didx_v.at[0])
        pltpu.async_copy(xall_hbm.at[sidx_v.at[0]], rows_v.at[0], gsems[0])

        @pl.loop(0, ng)
        def _(g):
            p = g % 2
            for j in range(G):
                b = j % 2
                if j + 1 < G:
                    if j == 0:
                        @pl.when(g > 0)
                        def _():
                            wait_scatter(1 - p, 1 - b)
                    else:
                        wait_scatter(p, 1 - b)
                    pltpu.async_copy(xall_hbm.at[sidx_v.at[j + 1]],
                                     rows_v.at[1 - b], gsems[1 - b])
                wait_gather(j, b)
                pltpu.async_copy(rows_v.at[b], acc.at[didx_v.at[p].at[j]],
                                 ssems[b], add=True)
                if j + 1 == G:
                    @pl.when(g + 1 < ng)
                    def _():
                        pltpu.sync_copy(
                            src_hbm.at[c].at[s].at[pl.ds((g + 1) * G, G)],
                            sidx_v)
                        pltpu.sync_copy(
                            dst_hbm.at[s].at[pl.ds((g + 1) * G, G)],
                            didx_v.at[1 - p])
                        wait_scatter(p, 1 - b)
                        pltpu.async_copy(xall_hbm.at[sidx_v.at[0]],
                                         rows_v.at[1 - b], gsems[1 - b])
                for q in range(CHUNK // L):
                    idx16 = didx_v[p, j, pl.ds(q * L, L)]
                    plsc.addupdate_scatter(dloc_v, [idx16], ones16)

        wait_scatter((ng - 1) % 2, (G - 2) % 2)
        wait_scatter((ng - 1) % 2, (G - 1) % 2)

        pltpu.sync_copy(dloc_v, parts_out.at[c * NS + s])
        plsc.subcore_barrier()

        for off, nrows in blks:
            pltpu.sync_copy(acc.at[pl.ds(base + off, nrows)],
                            rows_v.at[0].at[pl.ds(0, nrows)])
            pltpu.sync_copy(rows_v.at[0].at[pl.ds(0, nrows)],
                            sum_out.at[pl.ds(obase + off, nrows)])

    return k(xall, src_p, dst_p, zeros_feat)


def _tc_combine(x, summed, degs, wl0, wl1, wrt, b2, n, d_out, dh):
    bm = 1024
    grid = ((n + bm - 1) // bm,)

    def body(x_ref, s0_ref, s1_ref, p_ref,
             wl0_ref, wl1_ref, wrt_ref, b_ref, o_ref):
        deg = lax.dot_general(p_ref[...], jnp.ones((NS, 1), jnp.float32),
                              dimension_numbers=(((0,), (0,)), ((), ())),
                              preferred_element_type=jnp.float32)
        r = 1.0 / jnp.maximum(deg, 1.0)
        a0 = s0_ref[0] * r
        a1 = s1_ref[0] * r
        z = (jnp.dot(a0, wl0_ref[...], preferred_element_type=jnp.float32)
             + jnp.dot(a1, wl1_ref[...], preferred_element_type=jnp.float32)
             + jnp.dot(x_ref[...], wrt_ref[...],
                       preferred_element_type=jnp.float32)
             + b_ref[...])
        o_ref[...] = jnp.maximum(z, 0.0)

    return pl.pallas_call(
        body,
        grid=grid,
        in_specs=[
            pl.BlockSpec((bm, 2 * dh), lambda i: (i, 0)),
            pl.BlockSpec((1, bm, dh), lambda i: (0, i, 0)),
            pl.BlockSpec((1, bm, dh), lambda i: (1, i, 0)),
            pl.BlockSpec((NS, bm), lambda i: (0, i)),
            pl.BlockSpec((dh, d_out), lambda i: (0, 0)),
            pl.BlockSpec((dh, d_out), lambda i: (0, 0)),
            pl.BlockSpec((2 * dh, d_out), lambda i: (0, 0)),
            pl.BlockSpec((1, d_out), lambda i: (0, 0)),
        ],
        out_specs=pl.BlockSpec((bm, d_out), lambda i: (i, 0)),
        out_shape=jax.ShapeDtypeStruct((n, d_out), jnp.float32),
    )(x, summed, summed, degs, wl0, wl1, wrt, b2)


def kernel(x, edge_index, W_l, b_l, W_r):
    n, d_in = x.shape
    d_out = W_l.shape[0]
    dh = d_in // 2
    e = edge_index.shape[1]

    eps = ((e + NS * G * CHUNK - 1) // (NS * G * CHUNK)) * G * CHUNK
    e_pad = eps * NS
    cps = eps // CHUNK
    n_pad = ((n + 1 + NS * CHUNK - 1) // (NS * CHUNK)) * (NS * CHUNK)

    src = edge_index[0].astype(jnp.int32)
    dst = edge_index[1].astype(jnp.int32)
    pad = e_pad - e
    src_flat = jnp.concatenate([src, jnp.zeros((pad,), jnp.int32)])
    dst_flat = jnp.concatenate([dst, jnp.full((pad,), n, jnp.int32)])
    src_p = jnp.stack([2 * src_flat, 2 * src_flat + 1]).reshape(
        NC, NS, cps, CHUNK)
    dst_p = dst_flat.reshape(NS, cps, CHUNK)
    xview = x.reshape(NC * n, dh)

    zeros_feat = jnp.zeros((CHUNK, dh), jnp.float32)

    summed, parts = _sc_aggregate(xview, src_p, dst_p, zeros_feat,
                                  n_pad, cps, dh)
    summed = summed.reshape(NC, n_pad, dh)

    wl0 = W_l[:, :dh].T
    wl1 = W_l[:, dh:].T
    wrt = W_r.T
    b2 = b_l.reshape(1, d_out)
    return _tc_combine(x, summed, parts, wl0, wl1, wrt, b2, n, d_out, dh)


# speedup vs baseline: 1.0299x; 1.0236x over previous
import dataclasses
import functools

import jax
import jax.numpy as jnp
from jax import lax
from jax.experimental import pallas as pl
from jax.experimental.pallas import tpu as pltpu
from jax.experimental.pallas import tpu_sc as plsc

NC = 2
NS = 16
L = 16
CHUNK = 128
DEG_W = 16
G = 4


def _sc_aggregate(xall, src_p, dst_p, zeros_feat, n_pad, cps, dh):
    mesh = plsc.VectorSubcoreMesh(core_axis_name="c", subcore_axis_name="s")
    rpw = n_pad // NS

    full_blks, tail = rpw // CHUNK, rpw % CHUNK
    blks = [(k * CHUNK, CHUNK) for k in range(full_blks)]
    if tail:
        blks.append((full_blks * CHUNK, tail))

    cp = pltpu.CompilerParams()
    if "needs_layout_passes" in pltpu.CompilerParams.__dataclass_fields__:
        cp = dataclasses.replace(cp, needs_layout_passes=False)

    @functools.partial(
        pl.kernel,
        compiler_params=cp,
        out_type=[
            jax.ShapeDtypeStruct((NC * n_pad, dh), jnp.float32),
            jax.ShapeDtypeStruct((NC * NS, n_pad), jnp.float32),
        ],
        mesh=mesh,
        scratch_types=[
            pltpu.VMEM((G, CHUNK), jnp.int32),
            pltpu.VMEM((2, G, CHUNK), jnp.int32),
            pltpu.VMEM((2, CHUNK, dh), jnp.float32),
            pltpu.VMEM((n_pad,), jnp.float32),
            pltpu.VMEM_SHARED((n_pad, dh), jnp.float32),
            pltpu.SemaphoreType.DMA,
            pltpu.SemaphoreType.DMA,
            pltpu.SemaphoreType.DMA,
            pltpu.SemaphoreType.DMA,
        ],
    )
    def k(xall_hbm, src_hbm, dst_hbm, zf_hbm,
          sum_out, parts_out,
          sidx_v, didx_v, rows_v, dloc_v,
          acc, gsem0, gsem1, ssem0, ssem1):
        c = lax.axis_index("c")
        s = lax.axis_index("s")
        base = s * rpw
        obase = c * n_pad + s * rpw
        ones16 = jnp.full((L,), 1.0, jnp.float32)

        zero16 = jnp.zeros((L,), jnp.float32)

        @pl.loop(0, n_pad, step=L)
        def _(i):
            dloc_v[pl.ds(i, L)] = zero16

        pltpu.sync_copy(zf_hbm, rows_v.at[0])
        for off, nrows in blks:
            pltpu.sync_copy(rows_v.at[0].at[pl.ds(0, nrows)],
                            acc.at[pl.ds(base + off, nrows)])
        plsc.subcore_barrier()

        gsems = (gsem0, gsem1)
        ssems = (ssem0, ssem1)
        ng = cps // G

        def wait_gather(j, b):
            pltpu.make_async_copy(xall_hbm.at[sidx_v.at[j]],
                                  rows_v.at[b], gsems[b]).wait()

        def wait_scatter(p, b):
            pltpu.make_async_copy(rows_v.at[b],
                                  acc.at[didx_v.at[p].at[0]],
                                  ssems[b]).wait()

        pltpu.sync_copy(src_hbm.at[c].at[s].at[pl.ds(0, G)], sidx_v)
        pltpu.sync_copy(dst_hbm.at[s].at[pl.ds(0, G)], didx_v.at[0])
        pltpu.async_copy(xall_hbm.at[sidx_v.at[0]], rows_v.at[0], gsems[0])

        @pl.loop(0, ng)
        def _(g):
            p = g % 2
            for j in range(G):
                b = j % 2
                if j + 1 < G:
                    if j == 0:
                        @pl.when(g > 0)
                        def _():
                            wait_scatter(1 - p, 1 - b)
                    else:
                        wait_scatter(p, 1 - b)
                    pltpu.async_copy(xall_hbm.at[sidx_v.at[j + 1]],
                                     rows_v.at[1 - b], gsems[1 - b])
                wait_gather(j, b)
                pltpu.async_copy(rows_v.at[b], acc.at[didx_v.at[p].at[j]],
                                 ssems[b], add=True)
                if j + 1 == G:
                    @pl.when(g + 1 < ng)
                    def _():
                        pltpu.sync_copy(
                            src_hbm.at[c].at[s].at[pl.ds((g + 1) * G, G)],
                            sidx_v)
                        pltpu.sync_copy(
                            dst_hbm.at[s].at[pl.ds((g + 1) * G, G)],
                            didx_v.at[1 - p])
                        wait_scatter(p, 1 - b)
                        pltpu.async_copy(xall_hbm.at[sidx_v.at[0]],
                                         rows_v.at[1 - b], gsems[1 - b])
                for q in range(CHUNK // L):
                    idx16 = didx_v[p, j, pl.ds(q * L, L)]
                    plsc.addupdate_scatter(dloc_v, [idx16], ones16)

        wait_scatter((ng - 1) % 2, (G - 2) % 2)
        wait_scatter((ng - 1) % 2, (G - 1) % 2)

        pltpu.sync_copy(dloc_v, parts_out.at[c * NS + s])
        plsc.subcore_barrier()

        for off, nrows in blks:
            pltpu.sync_copy(acc.at[pl.ds(base + off, nrows)],
                            rows_v.at[0].at[pl.ds(0, nrows)])
            pltpu.sync_copy(rows_v.at[0].at[pl.ds(0, nrows)],
                            sum_out.at[pl.ds(obase + off, nrows)])

    return k(xall, src_p, dst_p, zeros_feat)


def _tc_combine(x, summed, degs, wl0, wl1, wrt, b2, n, d_out, dh):
    bm = 1024
    grid = ((n + bm - 1) // bm,)

    def body(x_ref, s0_ref, s1_ref, p_ref,
             wl0_ref, wl1_ref, wrt_ref, b_ref, o_ref):
        deg = lax.dot_general(p_ref[...], jnp.ones((NS, 1), jnp.float32),
                              dimension_numbers=(((0,), (0,)), ((), ())),
                              preferred_element_type=jnp.float32)
        r = 1.0 / jnp.maximum(deg, 1.0)
        a0 = s0_ref[0] * r
        a1 = s1_ref[0] * r
        z = (jnp.dot(a0, wl0_ref[...], preferred_element_type=jnp.float32)
             + jnp.dot(a1, wl1_ref[...], preferred_element_type=jnp.float32)
             + jnp.dot(x_ref[...], wrt_ref[...],
                       preferred_element_type=jnp.float32)
             + b_ref[...])
        o_ref[...] = jnp.maximum(z, 0.0)

    return pl.pallas_call(
        body,
        grid=grid,
        in_specs=[
            pl.BlockSpec((bm, 2 * dh), lambda i: (i, 0)),
            pl.BlockSpec((1, bm, dh), lambda i: (0, i, 0)),
            pl.BlockSpec((1, bm, dh), lambda i: (1, i, 0)),
            pl.BlockSpec((NS, bm), lambda i: (0, i)),
            pl.BlockSpec((dh, d_out), lambda i: (0, 0)),
            pl.BlockSpec((dh, d_out), lambda i: (0, 0)),
            pl.BlockSpec((2 * dh, d_out), lambda i: (0, 0)),
            pl.BlockSpec((1, d_out), lambda i: (0, 0)),
        ],
        out_specs=pl.BlockSpec((bm, d_out), lambda i: (i, 0)),
        out_shape=jax.ShapeDtypeStruct((n, d_out), jnp.float32),
    )(x, summed, summed, degs, wl0, wl1, wrt, b2)


def kernel(x, edge_index, W_l, b_l, W_r):
    n, d_in = x.shape
    d_out = W_l.shape[0]
    dh = d_in // 2
    e = edge_index.shape[1]

    eps = ((e + NS * G * CHUNK - 1) // (NS * G * CHUNK)) * G * CHUNK
    e_pad = eps * NS
    cps = eps // CHUNK
    n_pad = ((n + 1 + NS * CHUNK - 1) // (NS * CHUNK)) * (NS * CHUNK)

    src = edge_index[0].astype(jnp.int32)
    dst = edge_index[1].astype(jnp.int32)
    pad = e_pad - e
    src_flat = jnp.concatenate([src, jnp.zeros((pad,), jnp.int32)])
    dst_flat = jnp.concatenate([dst, jnp.full((pad,), n, jnp.int32)])
    src_p = jnp.stack([src_flat, src_flat + n]).reshape(NC, NS, cps, CHUNK)
    dst_p = dst_flat.reshape(NS, cps, CHUNK)
    xall = x.reshape(n, NC, dh).transpose(1, 0, 2).reshape(NC * n, dh)

    zeros_feat = jnp.zeros((CHUNK, dh), jnp.float32)

    summed, parts = _sc_aggregate(xall, src_p, dst_p, zeros_feat,
                                  n_pad, cps, dh)
    summed = summed.reshape(NC, n_pad, dh)

    wl0 = W_l[:, :dh].T
    wl1 = W_l[:, dh:].T
    wrt = W_r.T
    b2 = b_l.reshape(1, d_out)
    return _tc_combine(x, summed, parts, wl0, wl1, wrt, b2, n, d_out, dh)
